# Initial kernel scaffold; baseline (speedup 1.0000x reference)
import jax
import jax.numpy as jnp
from jax.experimental import pallas as pl
from jax.experimental.pallas import tpu as pltpu

SEQ = 1024
KC = 128
VOC = 2 * KC + 1
PADL = SEQ - KC
DIM = 64
BLOCK = 32


def _body(tk_ref, tv_ref, ok_ref, ov_ref, fk, fv):
    pid = pl.program_id(0)

    @pl.when(pid == 0)
    def _build():
        for t_ref, f in ((tk_ref, fk), (tv_ref, fv)):
            f[0:PADL, :] = jnp.broadcast_to(t_ref[0:1, :], (PADL, DIM))
            f[PADL:PADL + VOC - 1, :] = t_ref[0:VOC - 1, :]
            f[PADL + VOC - 1:2 * SEQ, :] = jnp.broadcast_to(
                t_ref[VOC - 1:VOC, :], (2 * SEQ - PADL - VOC + 1, DIM))

    for r in range(BLOCK):
        start = SEQ - pid * BLOCK - r
        ok_ref[r] = fk[pl.ds(start, SEQ), :]
        ov_ref[r] = fv[pl.ds(start, SEQ), :]


def kernel(seq_len, table_k, table_v):
    del seq_len
    out = pl.pallas_call(
        _body,
        grid=(SEQ // BLOCK,),
        in_specs=[
            pl.BlockSpec((VOC, DIM), lambda b: (0, 0)),
            pl.BlockSpec((VOC, DIM), lambda b: (0, 0)),
        ],
        out_specs=[
            pl.BlockSpec((BLOCK, SEQ, DIM), lambda b: (b, 0, 0)),
            pl.BlockSpec((BLOCK, SEQ, DIM), lambda b: (b, 0, 0)),
        ],
        out_shape=[
            jax.ShapeDtypeStruct((SEQ, SEQ, DIM), jnp.float32),
            jax.ShapeDtypeStruct((SEQ, SEQ, DIM), jnp.float32),
        ],
        scratch_shapes=[
            pltpu.VMEM((2 * SEQ, DIM), jnp.float32),
            pltpu.VMEM((2 * SEQ, DIM), jnp.float32),
        ],
    )(table_k, table_v)
    return (out[0], out[1])


# speedup vs baseline: 7.6092x; 7.6092x over previous
import jax
import jax.numpy as jnp
from jax.experimental import pallas as pl
from jax.experimental.pallas import tpu as pltpu

SEQ = 1024
KC = 128
VOC = 2 * KC + 1
DIM = 64
HB = SEQ // 2
PADP = (SEQ - KC) // 2
BODY = 2 * KC // 2
BLOCK = 32


def _body(pe_k, po_k, pe_v, po_v, t0p_k, t2p_k, t0p_v, t2p_v,
          ok_ref, ov_ref, fek, fok, fev, fov):
    pid = pl.program_id(0)

    @pl.when(pid == 0)
    def _build():
        for pe, po, t0p, t2p, fe, fo in (
                (pe_k, po_k, t0p_k, t2p_k, fek, fok),
                (pe_v, po_v, t0p_v, t2p_v, fev, fov)):
            head = jnp.broadcast_to(t0p[0:1, :], (PADP, 2 * DIM))
            tail = jnp.broadcast_to(t2p[0:1, :], (PADP, 2 * DIM))
            fe[0:PADP, :] = head
            fe[PADP:PADP + BODY, :] = pe[...]
            fe[PADP + BODY:2 * PADP + BODY, :] = tail
            fo[0:PADP, :] = head
            fo[PADP:PADP + BODY, :] = po[...]
            fo[PADP + BODY:2 * PADP + BODY, :] = tail

    for r in range(BLOCK):
        if r % 2 == 0:
            q = HB - pid * (BLOCK // 2) - r // 2
            ok_ref[r] = fek[pl.ds(q, HB), :]
            ov_ref[r] = fev[pl.ds(q, HB), :]
        else:
            q = HB - pid * (BLOCK // 2) - (r + 1) // 2
            ok_ref[r] = fok[pl.ds(q, HB), :]
            ov_ref[r] = fov[pl.ds(q, HB), :]


def kernel(seq_len, table_k, table_v):
    del seq_len
    pe_k = table_k[0:2 * KC].reshape(BODY, 2 * DIM)
    po_k = table_k[1:2 * KC + 1].reshape(BODY, 2 * DIM)
    pe_v = table_v[0:2 * KC].reshape(BODY, 2 * DIM)
    po_v = table_v[1:2 * KC + 1].reshape(BODY, 2 * DIM)
    t0p_k = jnp.tile(table_k[0:1], (1, 2))
    t2p_k = jnp.tile(table_k[2 * KC:2 * KC + 1], (1, 2))
    t0p_v = jnp.tile(table_v[0:1], (1, 2))
    t2p_v = jnp.tile(table_v[2 * KC:2 * KC + 1], (1, 2))

    full = lambda shape: pl.BlockSpec(shape, lambda b: tuple(0 for _ in shape))
    out = pl.pallas_call(
        _body,
        grid=(SEQ // BLOCK,),
        in_specs=[full((BODY, 2 * DIM))] * 4 + [full((1, 2 * DIM))] * 4,
        out_specs=[
            pl.BlockSpec((BLOCK, HB, 2 * DIM), lambda b: (b, 0, 0)),
            pl.BlockSpec((BLOCK, HB, 2 * DIM), lambda b: (b, 0, 0)),
        ],
        out_shape=[
            jax.ShapeDtypeStruct((SEQ, HB, 2 * DIM), jnp.float32),
            jax.ShapeDtypeStruct((SEQ, HB, 2 * DIM), jnp.float32),
        ],
        scratch_shapes=[pltpu.VMEM((SEQ, 2 * DIM), jnp.float32)] * 4,
    )(pe_k, po_k, pe_v, po_v, t0p_k, t2p_k, t0p_v, t2p_v)
    return (out[0].reshape(SEQ, SEQ, DIM), out[1].reshape(SEQ, SEQ, DIM))
